# baseline (device time: 114243 ns/iter reference)
import jax
import jax.numpy as jnp
from jax import lax
from jax.experimental import pallas as pl
from jax.experimental.pallas import tpu as pltpu

N_DEV = 16
SQ = 1024
SKV = 1024
DH = 128
H_LOC = 8
BLK = 64
CHUNK = SQ // N_DEV
SCALE = 0.08838834764831843


def _body(x_ref, wq_ref, k_ref, v_ref, wo_ref, out_ref,
          ctx_ref, part_ref, rs_buf,
          rs_send, rs_recv, ag_send, ag_recv):
    my = lax.axis_index("i")

    barrier = pltpu.get_barrier_semaphore()
    for j in range(1, N_DEV):
        tgt = lax.rem(my + j, N_DEV)
        pl.semaphore_signal(barrier, inc=1, device_id=(tgt,),
                            device_id_type=pl.DeviceIdType.MESH)
    pl.semaphore_wait(barrier, N_DEV - 1)

    kb = lax.broadcasted_iota(jnp.int32, (1, SKV), 1) // BLK

    def compute_chunk(qb):
        rows = pl.ds(qb * CHUNK, CHUNK)
        qs = lax.dot_general(
            x_ref[rows, :], wq_ref[:, :], (((1,), (0,)), ((), ())),
            preferred_element_type=jnp.float32).astype(jnp.bfloat16)
        keep = (kb == qb) | (kb == 0) | (lax.rem(kb + qb, 3) == 0)
        bias = jnp.where(keep, 0.0, -1e9)
        for h in range(H_LOC):
            sl = slice(h * DH, (h + 1) * DH)
            scores = lax.dot_general(
                qs[:, sl], k_ref[:, sl], (((1,), (1,)), ((), ())),
                preferred_element_type=jnp.float32)
            scores = scores * SCALE + bias
            mx = jnp.max(scores, axis=1, keepdims=True)
            e = jnp.exp(scores - mx)
            s = jnp.sum(e, axis=1, keepdims=True)
            w = (e / s).astype(jnp.bfloat16)
            ctx_ref[:, sl] = lax.dot_general(
                w, v_ref[:, sl], (((1,), (0,)), ((), ())),
                preferred_element_type=jnp.float32).astype(jnp.bfloat16)
        return lax.dot_general(
            ctx_ref[:, :], wo_ref[:, :], (((1,), (0,)), ((), ())),
            preferred_element_type=jnp.float32)

    rs_rdmas = []
    for j in range(1, N_DEV):
        tgt = lax.rem(my + j, N_DEV)
        rows = pl.ds(tgt * CHUNK, CHUNK)
        part_ref[rows, :] = compute_chunk(tgt).astype(jnp.bfloat16)
        rdma = pltpu.make_async_remote_copy(
            src_ref=part_ref.at[rows, :],
            dst_ref=rs_buf.at[j - 1],
            send_sem=rs_send.at[j - 1],
            recv_sem=rs_recv.at[j - 1],
            device_id=(tgt,),
            device_id_type=pl.DeviceIdType.MESH,
        )
        rdma.start()
        rs_rdmas.append(rdma)

    acc = compute_chunk(my)
    for k, rdma in enumerate(rs_rdmas):
        rdma.wait_recv()
        acc = acc + rs_buf[k, :, :].astype(jnp.float32)
    out_ref[pl.ds(my * CHUNK, CHUNK), :] = acc.astype(jnp.bfloat16)

    ag_rdmas = []
    for j in range(1, N_DEV):
        tgt = lax.rem(my + j, N_DEV)
        rdma = pltpu.make_async_remote_copy(
            src_ref=out_ref.at[pl.ds(my * CHUNK, CHUNK), :],
            dst_ref=out_ref.at[pl.ds(my * CHUNK, CHUNK), :],
            send_sem=ag_send.at[j - 1],
            recv_sem=ag_recv.at[j - 1],
            device_id=(tgt,),
            device_id_type=pl.DeviceIdType.MESH,
        )
        rdma.start()
        ag_rdmas.append(rdma)

    for k in range(N_DEV - 1):
        src_dev = lax.rem(my - k - 1 + N_DEV, N_DEV)
        recv = pltpu.make_async_remote_copy(
            src_ref=out_ref.at[pl.ds(my * CHUNK, CHUNK), :],
            dst_ref=out_ref.at[pl.ds(src_dev * CHUNK, CHUNK), :],
            send_sem=ag_send.at[k],
            recv_sem=ag_recv.at[k],
            device_id=(my,),
            device_id_type=pl.DeviceIdType.MESH,
        )
        recv.wait_recv()

    for rdma in rs_rdmas:
        rdma.wait_send()
    for rdma in ag_rdmas:
        rdma.wait_send()


def kernel(x, Wq, K_ext, V_ext, Wo):
    my = lax.axis_index("i")
    x2 = x.reshape(SQ, 1024).astype(jnp.bfloat16)
    wq = Wq.astype(jnp.bfloat16)
    k = lax.dynamic_slice(
        K_ext, (0, 0, my * H_LOC, 0), (1, SKV, H_LOC, DH)
    ).reshape(SKV, H_LOC * DH).astype(jnp.bfloat16)
    v = lax.dynamic_slice(
        V_ext, (0, 0, my * H_LOC, 0), (1, SKV, H_LOC, DH)
    ).reshape(SKV, H_LOC * DH).astype(jnp.bfloat16)
    wo = Wo.astype(jnp.bfloat16)

    out = pl.pallas_call(
        _body,
        out_shape=jax.ShapeDtypeStruct((SQ, 1024), jnp.bfloat16),
        in_specs=[pl.BlockSpec(memory_space=pltpu.VMEM)] * 5,
        out_specs=pl.BlockSpec(memory_space=pltpu.VMEM),
        scratch_shapes=[
            pltpu.VMEM((CHUNK, H_LOC * DH), jnp.bfloat16),
            pltpu.VMEM((SQ, 1024), jnp.bfloat16),
            pltpu.VMEM((N_DEV - 1, CHUNK, 1024), jnp.bfloat16),
            pltpu.SemaphoreType.DMA((N_DEV - 1,)),
            pltpu.SemaphoreType.DMA((N_DEV - 1,)),
            pltpu.SemaphoreType.DMA((N_DEV - 1,)),
            pltpu.SemaphoreType.DMA((N_DEV - 1,)),
        ],
        compiler_params=pltpu.CompilerParams(collective_id=0),
    )(x2, wq, k, v, wo)
    return out.reshape(1, SQ, 1024)


# device time: 71564 ns/iter; 1.5964x vs baseline; 1.5964x over previous
import jax
import jax.numpy as jnp
from jax import lax
from jax.experimental import pallas as pl
from jax.experimental.pallas import tpu as pltpu

N_DEV = 16
SQ = 1024
SKV = 1024
DH = 128
H_LOC = 8
BLK = 64
CHUNK = SQ // N_DEV
N_SLAB = 4
SLAB = SQ // N_SLAB
SCALE = 0.08838834764831843


def _body(x_ref, wq_ref, k_ref, v_ref, wo_ref, out_ref,
          ctx_ref, part_ref, rs_buf,
          rs_send, rs_recv, ag_send, ag_recv):
    my = lax.axis_index("i")

    barrier = pltpu.get_barrier_semaphore()
    for j in range(1, N_DEV):
        tgt = lax.rem(my + j, N_DEV)
        pl.semaphore_signal(barrier, inc=1, device_id=(tgt,),
                            device_id_type=pl.DeviceIdType.MESH)
    pl.semaphore_wait(barrier, N_DEV - 1)

    qb = lax.broadcasted_iota(jnp.int32, (SLAB, 1), 0) // BLK
    kb = lax.broadcasted_iota(jnp.int32, (1, SKV), 1) // BLK

    def compute_slab(s):
        rows = pl.ds(s * SLAB, SLAB)
        qs = lax.dot_general(
            x_ref[rows, :], wq_ref[:, :], (((1,), (0,)), ((), ())),
            preferred_element_type=jnp.float32).astype(jnp.bfloat16)
        qblk = qb + s * (SLAB // BLK)
        keep = (kb == qblk) | (kb == 0) | (lax.rem(kb + qblk, 3) == 0)
        bias = jnp.where(keep, 0.0, -1e9)
        for h in range(H_LOC):
            sl = slice(h * DH, (h + 1) * DH)
            scores = lax.dot_general(
                qs[:, sl], k_ref[:, sl], (((1,), (1,)), ((), ())),
                preferred_element_type=jnp.float32)
            e = jnp.exp(scores * SCALE + bias)
            d = jnp.sum(e, axis=1, keepdims=True)
            w = (e / d).astype(jnp.bfloat16)
            ctx_ref[:, sl] = lax.dot_general(
                w, v_ref[:, sl], (((1,), (0,)), ((), ())),
                preferred_element_type=jnp.float32).astype(jnp.bfloat16)
        part_ref[rows, :] = lax.dot_general(
            ctx_ref[:, :], wo_ref[:, :], (((1,), (0,)), ((), ())),
            preferred_element_type=jnp.float32).astype(jnp.bfloat16)

    rs_rdmas = []

    def send_chunk(c):
        slot = lax.rem(c - my + N_DEV, N_DEV) - 1
        rdma = pltpu.make_async_remote_copy(
            src_ref=part_ref.at[pl.ds(c * CHUNK, CHUNK), :],
            dst_ref=rs_buf.at[slot],
            send_sem=rs_send.at[slot],
            recv_sem=rs_recv.at[slot],
            device_id=(c,),
            device_id_type=pl.DeviceIdType.MESH,
        )
        rdma.start()
        rs_rdmas.append(rdma)

    mygrp = my // N_SLAB
    mypos = lax.rem(my, N_SLAB)
    for t in range(1, N_SLAB):
        s = lax.rem(mygrp + t, N_SLAB)
        compute_slab(s)
        for q in range(N_SLAB):
            send_chunk(s * N_SLAB + q)
    compute_slab(mygrp)
    for r in range(1, N_SLAB):
        send_chunk(mygrp * N_SLAB + lax.rem(mypos + r, N_SLAB))

    acc = part_ref[pl.ds(my * CHUNK, CHUNK), :].astype(jnp.float32)
    for k in range(N_DEV - 1):
        recv = pltpu.make_async_remote_copy(
            src_ref=part_ref.at[pl.ds(my * CHUNK, CHUNK), :],
            dst_ref=rs_buf.at[k],
            send_sem=rs_send.at[k],
            recv_sem=rs_recv.at[k],
            device_id=(my,),
            device_id_type=pl.DeviceIdType.MESH,
        )
        recv.wait_recv()
        acc = acc + rs_buf[k, :, :].astype(jnp.float32)
    out_ref[pl.ds(my * CHUNK, CHUNK), :] = acc.astype(jnp.bfloat16)

    ag_rdmas = []
    for j in range(1, N_DEV):
        tgt = lax.rem(my + j, N_DEV)
        rdma = pltpu.make_async_remote_copy(
            src_ref=out_ref.at[pl.ds(my * CHUNK, CHUNK), :],
            dst_ref=out_ref.at[pl.ds(my * CHUNK, CHUNK), :],
            send_sem=ag_send.at[j - 1],
            recv_sem=ag_recv.at[j - 1],
            device_id=(tgt,),
            device_id_type=pl.DeviceIdType.MESH,
        )
        rdma.start()
        ag_rdmas.append(rdma)

    for k in range(N_DEV - 1):
        src_dev = lax.rem(my - k - 1 + N_DEV, N_DEV)
        recv = pltpu.make_async_remote_copy(
            src_ref=out_ref.at[pl.ds(my * CHUNK, CHUNK), :],
            dst_ref=out_ref.at[pl.ds(src_dev * CHUNK, CHUNK), :],
            send_sem=ag_send.at[k],
            recv_sem=ag_recv.at[k],
            device_id=(my,),
            device_id_type=pl.DeviceIdType.MESH,
        )
        recv.wait_recv()

    for rdma in rs_rdmas:
        rdma.wait_send()
    for rdma in ag_rdmas:
        rdma.wait_send()


def kernel(x, Wq, K_ext, V_ext, Wo):
    my = lax.axis_index("i")
    x2 = x.reshape(SQ, 1024).astype(jnp.bfloat16)
    wq = Wq.astype(jnp.bfloat16)
    k = lax.dynamic_slice(
        K_ext, (0, 0, my * H_LOC, 0), (1, SKV, H_LOC, DH)
    ).reshape(SKV, H_LOC * DH).astype(jnp.bfloat16)
    v = lax.dynamic_slice(
        V_ext, (0, 0, my * H_LOC, 0), (1, SKV, H_LOC, DH)
    ).reshape(SKV, H_LOC * DH).astype(jnp.bfloat16)
    wo = Wo.astype(jnp.bfloat16)

    out = pl.pallas_call(
        _body,
        out_shape=jax.ShapeDtypeStruct((SQ, 1024), jnp.bfloat16),
        in_specs=[pl.BlockSpec(memory_space=pltpu.VMEM)] * 5,
        out_specs=pl.BlockSpec(memory_space=pltpu.VMEM),
        scratch_shapes=[
            pltpu.VMEM((SLAB, H_LOC * DH), jnp.bfloat16),
            pltpu.VMEM((SQ, 1024), jnp.bfloat16),
            pltpu.VMEM((N_DEV - 1, CHUNK, 1024), jnp.bfloat16),
            pltpu.SemaphoreType.DMA((N_DEV - 1,)),
            pltpu.SemaphoreType.DMA((N_DEV - 1,)),
            pltpu.SemaphoreType.DMA((N_DEV - 1,)),
            pltpu.SemaphoreType.DMA((N_DEV - 1,)),
        ],
        compiler_params=pltpu.CompilerParams(collective_id=0),
    )(x2, wq, k, v, wo)
    return out.reshape(1, SQ, 1024)
